# Initial kernel scaffold; baseline (speedup 1.0000x reference)
import functools

import jax
import jax.numpy as jnp
from jax import lax
from jax.experimental import pallas as pl
from jax.experimental.pallas import tpu as pltpu
from jax.experimental.pallas import tpu_sc as plsc

D_MODEL = 128
LANES = 16
NUM_WORKERS = 32
CHUNK = 128


@functools.partial(jax.jit, static_argnames=("n_rows",))
def _sc_gather_add(x2, y2, pe, n_rows):
    n_per_w = n_rows // NUM_WORKERS
    n_chunks = n_per_w // CHUNK

    mesh = plsc.VectorSubcoreMesh(core_axis_name="c", subcore_axis_name="s")

    @functools.partial(
        pl.kernel,
        out_type=jax.ShapeDtypeStruct((n_rows, D_MODEL), jnp.float32),
        mesh=mesh,
        scratch_types=[
            pltpu.VMEM((n_chunks, CHUNK), jnp.int32),
            pltpu.VMEM((CHUNK, D_MODEL), jnp.float32),
            pltpu.VMEM((CHUNK, D_MODEL), jnp.float32),
            pltpu.VMEM((CHUNK, D_MODEL), jnp.float32),
            pltpu.VMEM((CHUNK, D_MODEL), jnp.float32),
            pltpu.VMEM((CHUNK, D_MODEL), jnp.float32),
            pltpu.VMEM((CHUNK, D_MODEL), jnp.float32),
            pltpu.SemaphoreType.DMA,
            pltpu.SemaphoreType.DMA,
            pltpu.SemaphoreType.DMA,
            pltpu.SemaphoreType.DMA,
        ],
    )
    def run(x_hbm, y_hbm, pe_hbm, out_hbm,
            idx_v, xb0, xb1, pb0, pb1, ob0, ob1, is0, is1, os0, os1):
        xbufs, pbufs, obufs = (xb0, xb1), (pb0, pb1), (ob0, ob1)
        in_sems, out_sems = (is0, is1), (os0, os1)

        wid = lax.axis_index("s") * 2 + lax.axis_index("c")
        row0 = wid * n_per_w

        pltpu.sync_copy(y_hbm.at[pl.ds(wid * n_chunks, n_chunks)], idx_v)

        def start_in(j, b):
            pltpu.make_async_copy(
                x_hbm.at[pl.ds(row0 + j * CHUNK, CHUNK)], xbufs[b], in_sems[b]
            ).start()
            pltpu.make_async_copy(
                pe_hbm.at[idx_v.at[j]], pbufs[b], in_sems[b]
            ).start()

        def wait_in(b):
            pltpu.make_async_copy(
                x_hbm.at[pl.ds(0, CHUNK)], xbufs[b], in_sems[b]
            ).wait()
            pltpu.make_async_copy(
                x_hbm.at[pl.ds(0, CHUNK)], pbufs[b], in_sems[b]
            ).wait()

        def start_out(j, b):
            pltpu.make_async_copy(
                obufs[b], out_hbm.at[pl.ds(row0 + j * CHUNK, CHUNK)], out_sems[b]
            ).start()

        def wait_out(b):
            pltpu.make_async_copy(
                obufs[b], out_hbm.at[pl.ds(0, CHUNK)], out_sems[b]
            ).wait()

        def compute(b):
            xr, pr, outr = xbufs[b], pbufs[b], obufs[b]

            def row_body(r, carry):
                for c in range(D_MODEL // LANES):
                    sl = pl.ds(c * LANES, LANES)
                    outr[r, sl] = xr[r, sl] + pr[r, sl]
                return carry

            lax.fori_loop(0, CHUNK, row_body, 0, unroll=2)

        start_in(0, 0)
        start_in(1, 1)

        def outer(j0, carry):
            for b in range(2):
                j = j0 * 2 + b
                wait_in(b)

                @pl.when(j >= 2)
                def _():
                    wait_out(b)

                compute(b)
                start_out(j, b)

                @pl.when(j + 2 < n_chunks)
                def _():
                    start_in(j + 2, b)

            return carry

        lax.fori_loop(0, n_chunks // 2, outer, 0)
        wait_out(0)
        wait_out(1)

    return run(x2, y2, pe)


def kernel(x, y, pe):
    seq, batch, d = x.shape
    n_rows = seq * batch
    x2 = x.reshape(n_rows, d)
    y2 = y.reshape(-1).astype(jnp.int32).reshape(n_rows // CHUNK, CHUNK)
    out = _sc_gather_add(x2, y2, pe, n_rows)
    return out.reshape(x.shape)


# speedup vs baseline: 2.7348x; 2.7348x over previous
import functools

import jax
import jax.numpy as jnp
from jax import lax
from jax.experimental import pallas as pl
from jax.experimental.pallas import tpu as pltpu
from jax.experimental.pallas import tpu_sc as plsc

D_MODEL = 128
LANES = 16
NUM_WORKERS = 32
CHUNK = 128


@functools.partial(jax.jit, static_argnames=("n_rows",))
def _sc_gather_add(x2, y2, pe, n_rows):
    n_per_w = n_rows // NUM_WORKERS
    n_chunks = n_per_w // CHUNK

    mesh = plsc.VectorSubcoreMesh(core_axis_name="c", subcore_axis_name="s")

    @functools.partial(
        pl.kernel,
        out_type=jax.ShapeDtypeStruct((n_rows, D_MODEL), jnp.float32),
        mesh=mesh,
        scratch_types=[
            pltpu.VMEM((n_chunks, CHUNK), jnp.int32),
            pltpu.VMEM((CHUNK, D_MODEL), jnp.float32),
            pltpu.VMEM((CHUNK, D_MODEL), jnp.float32),
            pltpu.VMEM((CHUNK, D_MODEL), jnp.float32),
            pltpu.VMEM((CHUNK, D_MODEL), jnp.float32),
            pltpu.VMEM((CHUNK, D_MODEL), jnp.float32),
            pltpu.VMEM((CHUNK, D_MODEL), jnp.float32),
            pltpu.SemaphoreType.DMA,
            pltpu.SemaphoreType.DMA,
            pltpu.SemaphoreType.DMA,
            pltpu.SemaphoreType.DMA,
        ],
    )
    def run(x_hbm, y_hbm, pe_hbm, out_hbm,
            idx_v, xb0, xb1, pb0, pb1, ob0, ob1, is0, is1, os0, os1):
        xbufs, pbufs, obufs = (xb0, xb1), (pb0, pb1), (ob0, ob1)
        in_sems, out_sems = (is0, is1), (os0, os1)

        wid = lax.axis_index("s") * 2 + lax.axis_index("c")
        row0 = wid * n_per_w

        pltpu.sync_copy(y_hbm.at[wid], idx_v)

        def start_in(j, b):
            pltpu.make_async_copy(
                x_hbm.at[pl.ds(row0 + j * CHUNK, CHUNK)], xbufs[b], in_sems[b]
            ).start()
            pltpu.make_async_copy(
                pe_hbm.at[idx_v.at[j]], pbufs[b], in_sems[b]
            ).start()

        def wait_in(b):
            pltpu.make_async_copy(
                x_hbm.at[pl.ds(0, CHUNK)], xbufs[b], in_sems[b]
            ).wait()
            pltpu.make_async_copy(
                x_hbm.at[pl.ds(0, CHUNK)], pbufs[b], in_sems[b]
            ).wait()

        def start_out(j, b):
            pltpu.make_async_copy(
                obufs[b], out_hbm.at[pl.ds(row0 + j * CHUNK, CHUNK)], out_sems[b]
            ).start()

        def wait_out(b):
            pltpu.make_async_copy(
                obufs[b], out_hbm.at[pl.ds(0, CHUNK)], out_sems[b]
            ).wait()

        def compute(b):
            xr, pr, outr = xbufs[b], pbufs[b], obufs[b]

            def row_body(r, carry):
                for c in range(D_MODEL // LANES):
                    sl = pl.ds(c * LANES, LANES)
                    outr[r, sl] = xr[r, sl] + pr[r, sl]
                return carry

            lax.fori_loop(0, CHUNK, row_body, 0, unroll=2)

        start_in(0, 0)
        start_in(1, 1)

        def outer(j0, carry):
            for b in range(2):
                j = j0 * 2 + b
                wait_in(b)

                @pl.when(j >= 2)
                def _():
                    wait_out(b)

                compute(b)
                start_out(j, b)

                @pl.when(j + 2 < n_chunks)
                def _():
                    start_in(j + 2, b)

            return carry

        lax.fori_loop(0, n_chunks // 2, outer, 0)
        wait_out(0)
        wait_out(1)

    return run(x2, y2, pe)


def kernel(x, y, pe):
    seq, batch, d = x.shape
    n_rows = seq * batch
    x2 = x.reshape(n_rows, d)
    y2 = y.reshape(-1).astype(jnp.int32).reshape(
        NUM_WORKERS, n_rows // (NUM_WORKERS * CHUNK), CHUNK)
    out = _sc_gather_add(x2, y2, pe, n_rows)
    return out.reshape(x.shape)


# speedup vs baseline: 5.9841x; 2.1881x over previous
import functools

import jax
import jax.numpy as jnp
from jax import lax
from jax.experimental import pallas as pl
from jax.experimental.pallas import tpu as pltpu
from jax.experimental.pallas import tpu_sc as plsc

D_MODEL = 128
LANES = 16
NUM_WORKERS = 32
CHUNK = 128


@functools.partial(jax.jit, static_argnames=("n_rows",))
def _sc_gather_add(x2, y2, pe, n_rows):
    n_per_w = n_rows // NUM_WORKERS
    n_chunks = n_per_w // CHUNK

    mesh = plsc.VectorSubcoreMesh(core_axis_name="c", subcore_axis_name="s")

    @functools.partial(
        pl.kernel,
        out_type=jax.ShapeDtypeStruct((n_rows, D_MODEL), jnp.float32),
        mesh=mesh,
        scratch_types=[
            pltpu.VMEM((n_chunks, CHUNK), jnp.int32),
            pltpu.VMEM((CHUNK, D_MODEL), jnp.float32),
            pltpu.VMEM((CHUNK, D_MODEL), jnp.float32),
            pltpu.VMEM((CHUNK, D_MODEL), jnp.float32),
            pltpu.VMEM((CHUNK, D_MODEL), jnp.float32),
            pltpu.VMEM((CHUNK, D_MODEL), jnp.float32),
            pltpu.VMEM((CHUNK, D_MODEL), jnp.float32),
            pltpu.SemaphoreType.DMA,
            pltpu.SemaphoreType.DMA,
            pltpu.SemaphoreType.DMA,
            pltpu.SemaphoreType.DMA,
        ],
    )
    def run(x_hbm, y_hbm, pe_hbm, out_hbm,
            idx_v, xb0, xb1, pb0, pb1, ob0, ob1, is0, is1, os0, os1):
        xbufs, pbufs, obufs = (xb0, xb1), (pb0, pb1), (ob0, ob1)
        in_sems, out_sems = (is0, is1), (os0, os1)

        wid = lax.axis_index("s") * 2 + lax.axis_index("c")
        row0 = wid * n_per_w

        pltpu.sync_copy(y_hbm.at[wid], idx_v)

        def start_in(j, b):
            pltpu.make_async_copy(
                x_hbm.at[pl.ds(row0 + j * CHUNK, CHUNK)], xbufs[b], in_sems[b]
            ).start()
            pltpu.make_async_copy(
                pe_hbm.at[idx_v.at[j]], pbufs[b], in_sems[b]
            ).start()

        def wait_in(b):
            pltpu.make_async_copy(
                x_hbm.at[pl.ds(0, CHUNK)], xbufs[b], in_sems[b]
            ).wait()
            pltpu.make_async_copy(
                x_hbm.at[pl.ds(0, CHUNK)], pbufs[b], in_sems[b]
            ).wait()

        def start_out(j, b):
            pltpu.make_async_copy(
                obufs[b], out_hbm.at[pl.ds(row0 + j * CHUNK, CHUNK)], out_sems[b]
            ).start()

        def wait_out(b):
            pltpu.make_async_copy(
                obufs[b], out_hbm.at[pl.ds(0, CHUNK)], out_sems[b]
            ).wait()

        def compute(b):
            xr, pr, outr = xbufs[b], pbufs[b], obufs[b]

            @plsc.parallel_loop(0, CHUNK, unroll=4)
            def row_body(r):
                for c in range(D_MODEL // LANES):
                    sl = pl.ds(c * LANES, LANES)
                    outr[r, sl] = xr[r, sl] + pr[r, sl]

        start_in(0, 0)
        start_in(1, 1)

        def outer(j0, carry):
            for b in range(2):
                j = j0 * 2 + b
                wait_in(b)

                @pl.when(j >= 2)
                def _():
                    wait_out(b)

                compute(b)
                start_out(j, b)

                @pl.when(j + 2 < n_chunks)
                def _():
                    start_in(j + 2, b)

            return carry

        lax.fori_loop(0, n_chunks // 2, outer, 0)
        wait_out(0)
        wait_out(1)

    return run(x2, y2, pe)


def kernel(x, y, pe):
    seq, batch, d = x.shape
    n_rows = seq * batch
    x2 = x.reshape(n_rows, d)
    y2 = y.reshape(-1).astype(jnp.int32).reshape(
        NUM_WORKERS, n_rows // (NUM_WORKERS * CHUNK), CHUNK)
    out = _sc_gather_add(x2, y2, pe, n_rows)
    return out.reshape(x.shape)
